# Initial kernel scaffold; baseline (speedup 1.0000x reference)
import jax
import jax.numpy as jnp
from jax.experimental import pallas as pl


def kernel(x, pos, edge_index, edge_attr, params, pos_scale_logit):
    raise NotImplementedError("write your pallas kernel here")


# speedup vs baseline: 3.7844x; 3.7844x over previous
import functools

import jax
import jax.numpy as jnp
from jax import lax
from jax.experimental import pallas as pl
from jax.experimental.pallas import tpu as pltpu
from jax.experimental.pallas import tpu_sc as plsc

N, E, D, H, P, ED = 10000, 320000, 128, 128, 3, 16

NC = 2
NS = 16
NW = NC * NS
E_PER_W = E // NW
CHUNK = 200
NCHUNK = E_PER_W // CHUNK
REL_CHUNK = 400
NREL = E_PER_W // REL_CHUNK
N_PAD = 10240
ROWS_PER_TILE = N_PAD // NS

_f32 = jnp.float32

BE = 3200
BN = 2000


def _sigmoid(v):
    return 1.0 / (1.0 + jnp.exp(-v))


def _silu(v):
    return v * _sigmoid(v)


_SC_MESH = plsc.VectorSubcoreMesh(core_axis_name="c", subcore_axis_name="s",
                                  num_cores=NC, num_subcores=NS)


@functools.partial(
    pl.kernel,
    out_type=jax.ShapeDtypeStruct((E, 16), _f32),
    mesh=_SC_MESH,
    compiler_params=pltpu.CompilerParams(needs_layout_passes=False),
    scratch_types=[
        pltpu.VMEM((P * N,), _f32),
        pltpu.VMEM((E_PER_W,), jnp.int32),
        pltpu.VMEM((E_PER_W,), jnp.int32),
        pltpu.VMEM((REL_CHUNK, 16), _f32),
    ],
)
def _sc_rel(posf_hbm, dst_hbm, src_hbm, rel_hbm, posv, di, si, rbuf):
    wid = lax.axis_index("c") * NS + lax.axis_index("s")
    base0 = wid * E_PER_W
    pltpu.sync_copy(posf_hbm, posv)
    pltpu.sync_copy(dst_hbm.at[pl.ds(base0, E_PER_W)], di)
    pltpu.sync_copy(src_hbm.at[pl.ds(base0, E_PER_W)], si)
    zero16 = jnp.zeros((16,), _f32)

    def zrow(r, carry):
        rbuf[r, :] = zero16
        return carry

    lax.fori_loop(0, REL_CHUNK, zrow, 0)
    lanes = lax.iota(jnp.int32, 16)

    def chunk(j, carry):
        def vec(v, carry2):
            e0 = j * REL_CHUNK + v * 16
            dstv = di[pl.ds(e0, 16)]
            srcv = si[pl.ds(e0, 16)]
            rows = v * 16 + lanes
            for comp in range(P):
                pdc = plsc.load_gather(posv, [dstv + comp * N])
                psc = plsc.load_gather(posv, [srcv + comp * N])
                cols = jnp.full((16,), comp, jnp.int32)
                plsc.store_scatter(rbuf, [rows, cols], pdc - psc)
            return carry2

        lax.fori_loop(0, REL_CHUNK // 16, vec, 0)
        pltpu.sync_copy(rbuf, rel_hbm.at[pl.ds(base0 + j * REL_CHUNK, REL_CHUNK)])
        return carry

    lax.fori_loop(0, NREL, chunk, 0)


@functools.partial(
    pl.kernel,
    out_type=jax.ShapeDtypeStruct((E, H), _f32),
    mesh=_SC_MESH,
    compiler_params=pltpu.CompilerParams(needs_layout_passes=False),
    scratch_types=[
        pltpu.VMEM((CHUNK,), jnp.int32),
        pltpu.VMEM((CHUNK,), jnp.int32),
        pltpu.VMEM((CHUNK, H), _f32),
        pltpu.VMEM((CHUNK, H), _f32),
        pltpu.SemaphoreType.DMA,
    ],
)
def _sc_gather(xd_hbm, xs_hbm, dst_hbm, src_hbm,
               pre0_hbm, di, si, bufd, bufs, sem):
    wid = lax.axis_index("c") * NS + lax.axis_index("s")
    base0 = wid * E_PER_W

    def chunk(j, carry):
        base = base0 + j * CHUNK
        pltpu.sync_copy(dst_hbm.at[pl.ds(base, CHUNK)], di)
        pltpu.sync_copy(src_hbm.at[pl.ds(base, CHUNK)], si)
        cp1 = pltpu.async_copy(xd_hbm.at[di], bufd, sem)
        cp2 = pltpu.async_copy(xs_hbm.at[si], bufs, sem)
        cp1.wait()
        cp2.wait()

        def addrow(r, carry2):
            for k in range(H // 16):
                sl = pl.ds(k * 16, 16)
                bufd[r, sl] = bufd[r, sl] + bufs[r, sl]
            return carry2

        lax.fori_loop(0, CHUNK, addrow, 0)
        pltpu.sync_copy(bufd, pre0_hbm.at[pl.ds(base, CHUNK)])
        return carry

    lax.fori_loop(0, NCHUNK, chunk, 0)


@functools.partial(
    pl.kernel,
    out_type=jax.ShapeDtypeStruct((NC, N_PAD, H), _f32),
    mesh=_SC_MESH,
    compiler_params=pltpu.CompilerParams(needs_layout_passes=False),
    scratch_types=[
        pltpu.VMEM_SHARED((N_PAD, H), _f32),
        pltpu.VMEM((CHUNK,), jnp.int32),
        pltpu.VMEM((CHUNK, H), _f32),
    ],
)
def _sc_scatter(m_hbm, dst_hbm, z_hbm, agg_hbm, shared, di, mbuf):
    c = lax.axis_index("c")
    s = lax.axis_index("s")
    wid = c * NS + s
    rows = pl.ds(s * ROWS_PER_TILE, ROWS_PER_TILE)
    pltpu.sync_copy(z_hbm.at[rows], shared.at[rows])
    plsc.subcore_barrier()

    def chunk(j, carry):
        base = wid * E_PER_W + j * CHUNK
        pltpu.sync_copy(dst_hbm.at[pl.ds(base, CHUNK)], di)
        pltpu.sync_copy(m_hbm.at[pl.ds(base, CHUNK)], mbuf)
        pltpu.sync_copy(mbuf, shared.at[di], add=True)
        return carry

    lax.fori_loop(0, NCHUNK, chunk, 0)
    plsc.subcore_barrier()
    pltpu.sync_copy(shared.at[rows], agg_hbm.at[c, rows])


def _w_spec(shape):
    return pl.BlockSpec(shape, lambda i: (0,) * len(shape))


def _proj_body(x_ref, wd_ref, ws_ref, xd_ref, xs_ref):
    v = x_ref[...]
    xd_ref[...] = jnp.dot(v, wd_ref[...], preferred_element_type=_f32)
    xs_ref[...] = jnp.dot(v, ws_ref[...], preferred_element_type=_f32)


def _tc_proj(x, wd, ws):
    return pl.pallas_call(
        _proj_body,
        grid=(N // BN,),
        in_specs=[
            pl.BlockSpec((BN, D), lambda i: (i, 0)),
            _w_spec((D, H)),
            _w_spec((D, H)),
        ],
        out_specs=[
            pl.BlockSpec((BN, H), lambda i: (i, 0)),
            pl.BlockSpec((BN, H), lambda i: (i, 0)),
        ],
        out_shape=[
            jax.ShapeDtypeStruct((N, H), _f32),
            jax.ShapeDtypeStruct((N, H), _f32),
        ],
    )(x, wd, ws)


def _edge1_body(pre0_ref, rel_ref, ea_ref, wdist_ref, we_ref, b0_ref,
                w1_ref, b1_ref, lmask_ref, m_ref):
    rel = rel_ref[...]
    dist2 = jnp.sum(rel * rel * lmask_ref[...], axis=-1, keepdims=True)
    pre = (pre0_ref[...] + dist2 * wdist_ref[...] + b0_ref[...]
           + jnp.dot(ea_ref[...], we_ref[...], preferred_element_type=_f32))
    m1 = _silu(pre)
    z = jnp.dot(m1, w1_ref[...], preferred_element_type=_f32) + b1_ref[...]
    m_ref[...] = _silu(z)


def _tc_edge1(pre0, rel, ea, wdist, we, b0, w1, b1, lmask):
    return pl.pallas_call(
        _edge1_body,
        grid=(E // BE,),
        in_specs=[
            pl.BlockSpec((BE, H), lambda i: (i, 0)),
            pl.BlockSpec((BE, 16), lambda i: (i, 0)),
            pl.BlockSpec((BE, ED), lambda i: (i, 0)),
            _w_spec((1, H)),
            _w_spec((ED, H)),
            _w_spec((1, H)),
            _w_spec((H, H)),
            _w_spec((1, H)),
            _w_spec((1, 16)),
        ],
        out_specs=pl.BlockSpec((BE, H), lambda i: (i, 0)),
        out_shape=jax.ShapeDtypeStruct((E, H), _f32),
    )(pre0, rel, ea, wdist, we, b0, w1, b1, lmask)


def _edge2_body(pre0_ref, rel_ref, ea_ref, wdist_ref, we_ref, b0_ref,
                w1_ref, b1_ref, wp0_ref, bp0_ref, wp1_ref, bp1_ref,
                lmask_ref, oh3_ref, m_ref, relw_ref):
    rel = rel_ref[...]
    dist2 = jnp.sum(rel * rel * lmask_ref[...], axis=-1, keepdims=True)
    pre = (pre0_ref[...] + dist2 * wdist_ref[...] + b0_ref[...]
           + jnp.dot(ea_ref[...], we_ref[...], preferred_element_type=_f32))
    m1 = _silu(pre)
    z = jnp.dot(m1, w1_ref[...], preferred_element_type=_f32) + b1_ref[...]
    m = _silu(z)
    m_ref[...] = m
    t = jnp.dot(m, wp0_ref[...], preferred_element_type=_f32) + bp0_ref[...]
    t = _silu(t)
    w2 = jnp.sum(t * wp1_ref[...], axis=-1, keepdims=True) + bp1_ref[:, :1]
    relw = jnp.concatenate([rel * w2, jnp.zeros((BE, H - 16), _f32)], axis=1)
    relw_ref[...] = relw + oh3_ref[...]


def _tc_edge2(pre0, rel, ea, wdist, we, b0, w1, b1, wp0, bp0, wp1, bp1,
              lmask, oh3):
    return pl.pallas_call(
        _edge2_body,
        grid=(E // BE,),
        in_specs=[
            pl.BlockSpec((BE, H), lambda i: (i, 0)),
            pl.BlockSpec((BE, 16), lambda i: (i, 0)),
            pl.BlockSpec((BE, ED), lambda i: (i, 0)),
            _w_spec((1, H)),
            _w_spec((ED, H)),
            _w_spec((1, H)),
            _w_spec((H, H)),
            _w_spec((1, H)),
            _w_spec((H, H)),
            _w_spec((1, H)),
            _w_spec((1, H)),
            _w_spec((1, H)),
            _w_spec((1, 16)),
            _w_spec((1, H)),
        ],
        out_specs=[
            pl.BlockSpec((BE, H), lambda i: (i, 0)),
            pl.BlockSpec((BE, H), lambda i: (i, 0)),
        ],
        out_shape=[
            jax.ShapeDtypeStruct((E, H), _f32),
            jax.ShapeDtypeStruct((E, H), _f32),
        ],
    )(pre0, rel, ea, wdist, we, b0, w1, b1, wp0, bp0, wp1, bp1, lmask, oh3)


def _node1_body(x_ref, aggp_ref, wn0x_ref, wn0a_ref, bn0_ref, wn1_ref,
                bn1_ref, wd2_ref, ws2_ref, h_ref, xd2_ref, xs2_ref):
    agg = aggp_ref[0] + aggp_ref[1]
    t = (jnp.dot(x_ref[...], wn0x_ref[...], preferred_element_type=_f32)
         + jnp.dot(agg, wn0a_ref[...], preferred_element_type=_f32)
         + bn0_ref[...])
    t = _silu(t)
    hv = jnp.dot(t, wn1_ref[...], preferred_element_type=_f32) + bn1_ref[...]
    h_ref[...] = hv
    xd2_ref[...] = jnp.dot(hv, wd2_ref[...], preferred_element_type=_f32)
    xs2_ref[...] = jnp.dot(hv, ws2_ref[...], preferred_element_type=_f32)


def _tc_node1(x, aggp, wn0x, wn0a, bn0, wn1, bn1, wd2, ws2):
    return pl.pallas_call(
        _node1_body,
        grid=(N // BN,),
        in_specs=[
            pl.BlockSpec((BN, D), lambda i: (i, 0)),
            pl.BlockSpec((NC, BN, H), lambda i: (0, i, 0)),
            _w_spec((D, H)),
            _w_spec((H, H)),
            _w_spec((1, H)),
            _w_spec((H, H)),
            _w_spec((1, H)),
            _w_spec((H, H)),
            _w_spec((H, H)),
        ],
        out_specs=[
            pl.BlockSpec((BN, H), lambda i: (i, 0)),
            pl.BlockSpec((BN, H), lambda i: (i, 0)),
            pl.BlockSpec((BN, H), lambda i: (i, 0)),
        ],
        out_shape=[
            jax.ShapeDtypeStruct((N, H), _f32),
            jax.ShapeDtypeStruct((N, H), _f32),
            jax.ShapeDtypeStruct((N, H), _f32),
        ],
    )(x, aggp, wn0x, wn0a, bn0, wn1, bn1, wd2, ws2)


def _node2_body(h_ref, aggp_ref, pos_ref, pacc_ref, logit_ref, wn0x_ref,
                wn0a_ref, bn0_ref, wn1_ref, bn1_ref, lmask_ref, oh3_ref,
                xout_ref, posout_ref):
    agg = aggp_ref[0] + aggp_ref[1]
    t = (jnp.dot(h_ref[...], wn0x_ref[...], preferred_element_type=_f32)
         + jnp.dot(agg, wn0a_ref[...], preferred_element_type=_f32)
         + bn0_ref[...])
    t = _silu(t)
    xout_ref[...] = (jnp.dot(t, wn1_ref[...], preferred_element_type=_f32)
                     + bn1_ref[...])
    acc = pacc_ref[0] + pacc_ref[1]
    deg = jnp.sum(acc * oh3_ref[...], axis=-1, keepdims=True)
    msg = acc * lmask_ref[...]
    gate = _sigmoid(logit_ref[...])
    upd = jnp.clip(gate * msg / jnp.maximum(deg, 1.0), -5.0, 5.0)
    posout_ref[...] = jnp.clip(pos_ref[...] + upd, -500.0, 500.0)


def _tc_node2(h, aggp, pos16, pacc, logit16, wn0x, wn0a, bn0, wn1, bn1,
              lmask, oh3):
    return pl.pallas_call(
        _node2_body,
        grid=(N // BN,),
        in_specs=[
            pl.BlockSpec((BN, H), lambda i: (i, 0)),
            pl.BlockSpec((NC, BN, H), lambda i: (0, i, 0)),
            pl.BlockSpec((BN, H), lambda i: (i, 0)),
            pl.BlockSpec((NC, BN, H), lambda i: (0, i, 0)),
            _w_spec((1, H)),
            _w_spec((H, H)),
            _w_spec((H, H)),
            _w_spec((1, H)),
            _w_spec((H, H)),
            _w_spec((1, H)),
            _w_spec((1, H)),
            _w_spec((1, H)),
        ],
        out_specs=[
            pl.BlockSpec((BN, H), lambda i: (i, 0)),
            pl.BlockSpec((BN, H), lambda i: (i, 0)),
        ],
        out_shape=[
            jax.ShapeDtypeStruct((N, H), _f32),
            jax.ShapeDtypeStruct((N, H), _f32),
        ],
    )(h, aggp, pos16, pacc, logit16, wn0x, wn0a, bn0, wn1, bn1, lmask, oh3)


def kernel(x, pos, edge_index, edge_attr, params, pos_scale_logit):
    src = edge_index[0]
    dst = edge_index[1]
    pos128 = jnp.zeros((N, H), _f32).at[:, :P].set(pos)
    z128 = jnp.zeros((N_PAD, H), _f32)
    lmask = jnp.zeros((1, 16), _f32).at[0, :P].set(1.0)
    lmask128 = jnp.zeros((1, H), _f32).at[0, :P].set(1.0)
    oh3 = jnp.zeros((1, H), _f32).at[0, P].set(1.0)
    logit128 = jnp.full((1, H), pos_scale_logit, _f32)

    lp1, lp2 = params

    def edge_w(lp):
        w0, b0 = lp['edge0']
        return (w0[:D], w0[D:2 * D], w0[2 * D:2 * D + 1], w0[2 * D + 1:],
                b0.reshape(1, H))

    wd1, ws1, wdist1, we1, b01 = edge_w(lp1)
    wd2, ws2, wdist2, we2, b02 = edge_w(lp2)
    w11, b11 = lp1['edge1'][0], lp1['edge1'][1].reshape(1, H)
    w12, b12 = lp2['edge1'][0], lp2['edge1'][1].reshape(1, H)
    wn0x1, wn0a1 = lp1['node0'][0][:D], lp1['node0'][0][D:]
    bn01 = lp1['node0'][1].reshape(1, H)
    wn11, bn11 = lp1['node1'][0], lp1['node1'][1].reshape(1, H)
    wn0x2, wn0a2 = lp2['node0'][0][:H], lp2['node0'][0][H:]
    bn02 = lp2['node0'][1].reshape(1, H)
    wn12, bn12 = lp2['node1'][0], lp2['node1'][1].reshape(1, H)
    wp0, bp0 = lp2['pos0'][0], lp2['pos0'][1].reshape(1, H)
    wp1 = lp2['pos1'][0].reshape(1, H)
    bp1 = jnp.broadcast_to(lp2['pos1'][1].reshape(1, 1), (1, H))

    posf = pos.T.reshape(-1)
    rel = _sc_rel(posf, dst, src)
    xd1, xs1 = _tc_proj(x, wd1, ws1)
    pre0_1 = _sc_gather(xd1, xs1, dst, src)
    m1 = _tc_edge1(pre0_1, rel, edge_attr, wdist1, we1, b01, w11, b11, lmask)
    aggp1 = _sc_scatter(m1, dst, z128)
    h, xd2, xs2 = _tc_node1(x, aggp1, wn0x1, wn0a1, bn01, wn11, bn11,
                            wd2, ws2)

    pre0_2 = _sc_gather(xd2, xs2, dst, src)
    m2, relw = _tc_edge2(pre0_2, rel, edge_attr, wdist2, we2, b02, w12, b12,
                         wp0, bp0, wp1, bp1, lmask, oh3)
    aggp2 = _sc_scatter(m2, dst, z128)
    pacc = _sc_scatter(relw, dst, z128)
    x_out, pos_out = _tc_node2(h, aggp2, pos128, pacc, logit128, wn0x2,
                               wn0a2, bn02, wn12, bn12, lmask128, oh3)
    return x_out, pos_out[:, :P]
